# baseline (speedup 1.0000x reference)
import jax
import jax.numpy as jnp
from jax.experimental import pallas as pl

BLOCK_B = 1024
BATCH = 16384
NUM_FEATURES = 26
EMB_DIM = 64


def _tc_body(x_ref, w_ref, out_ref, norm_ref):
    w = w_ref[...]
    e = jnp.exp(w)
    s = jnp.sum(e, axis=0, keepdims=True)
    n = e / s
    norm_ref[...] = n.reshape(NUM_FEATURES, BLOCK_B // 128, 128)
    x = x_ref[...]
    out_ref[...] = jnp.sum(x * n[:, None, :], axis=0)


def _tc_pool(xt, gathered_t, block_b=1024):
    nb = BATCH // block_b
    out_shapes = (
        jax.ShapeDtypeStruct((EMB_DIM, BATCH), jnp.float32),
        jax.ShapeDtypeStruct((NUM_FEATURES, BATCH // 128, 128), jnp.float32),
    )
    return pl.pallas_call(
        _tc_body,
        grid=(nb,),
        in_specs=[
            pl.BlockSpec((NUM_FEATURES, EMB_DIM, block_b), lambda i: (0, 0, i)),
            pl.BlockSpec((NUM_FEATURES, block_b), lambda i: (0, i)),
        ],
        out_specs=(
            pl.BlockSpec((EMB_DIM, block_b), lambda i: (0, i)),
            pl.BlockSpec((NUM_FEATURES, block_b // 128, 128),
                         lambda i: (0, i, 0)),
        ),
        out_shape=out_shapes,
    )(xt, gathered_t)


@jax.jit
def kernel(inputs, item_indices, attention_weights):
    xt = jnp.transpose(inputs, (1, 2, 0))
    g = jax.lax.gather(
        attention_weights, item_indices[:, None],
        jax.lax.GatherDimensionNumbers(
            offset_dims=(1,), collapsed_slice_dims=(0,), start_index_map=(0,)),
        slice_sizes=(1, NUM_FEATURES),
        mode=jax.lax.GatherScatterMode.PROMISE_IN_BOUNDS)
    out_t, norm3 = _tc_pool(xt, g.T)
    norm_t = norm3.reshape(NUM_FEATURES, BATCH)
    return out_t.T, norm_t.T[:, :, None]


# speedup vs baseline: 1.0164x; 1.0164x over previous
import jax
import jax.numpy as jnp
from jax.experimental import pallas as pl

BATCH = 16384
NUM_FEATURES = 26
EMB_DIM = 64


def _tc_body(x_ref, w_ref, out_ref, norm_ref):
    w = w_ref[...]
    e = jnp.exp(w)
    s = jnp.sum(e, axis=0, keepdims=True)
    n = e / s
    norm_ref[...] = n
    x = x_ref[...]
    out_ref[...] = jnp.sum(x * n[:, None, :], axis=0)


def _tc_pool(xt, gathered_t, block_b=1024):
    nb = BATCH // block_b
    out_shapes = (
        jax.ShapeDtypeStruct((EMB_DIM, BATCH), jnp.float32),
        jax.ShapeDtypeStruct((NUM_FEATURES, BATCH), jnp.float32),
    )
    return pl.pallas_call(
        _tc_body,
        grid=(nb,),
        in_specs=[
            pl.BlockSpec((NUM_FEATURES, EMB_DIM, block_b), lambda i: (0, 0, i)),
            pl.BlockSpec((NUM_FEATURES, block_b), lambda i: (0, i)),
        ],
        out_specs=(
            pl.BlockSpec((EMB_DIM, block_b), lambda i: (0, i)),
            pl.BlockSpec((NUM_FEATURES, block_b), lambda i: (0, i)),
        ),
        out_shape=out_shapes,
    )(xt, gathered_t)


@jax.jit
def kernel(inputs, item_indices, attention_weights):
    xt = jnp.transpose(inputs, (1, 2, 0))
    g = jax.lax.gather(
        attention_weights, item_indices[:, None],
        jax.lax.GatherDimensionNumbers(
            offset_dims=(1,), collapsed_slice_dims=(0,), start_index_map=(0,)),
        slice_sizes=(1, NUM_FEATURES),
        mode=jax.lax.GatherScatterMode.PROMISE_IN_BOUNDS)
    out_t, norm_t = _tc_pool(xt, g.T)
    return out_t.T, norm_t.T[:, :, None]
